# initial kernel scaffold (unmeasured)
import jax
import jax.numpy as jnp
from jax import lax
from jax.experimental import pallas as pl
from jax.experimental.pallas import tpu as pltpu

N_DEV = 16
B, Sq, Hq, Dh = 2, 256, 4, 64
BLK = 64
NQB = Sq // BLK
DM = Hq * Dh
BF = jnp.bfloat16
F32 = jnp.float32


def kernel(x, Wq, K_ext, V_ext, Wo):
    d_model = x.shape[-1]

    def body(x_ref, wq_ref, k_ref, v_ref, wo_ref, out_ref,
             acc_tx, l_tx, acc_rx, l_rx,
             acc_send, acc_recv, l_send, l_recv):
        me = lax.axis_index("i")

        for b in range(B):
            q_full = jnp.dot(x_ref[b].astype(BF), wq_ref[...].astype(BF),
                             preferred_element_type=F32)
            for h in range(Hq):
                for qb in range(NQB):
                    rows = slice(qb * BLK, (qb + 1) * BLK)
                    qblk = q_full[rows, h * Dh:(h + 1) * Dh].astype(BF)
                    kblk = k_ref[b, rows, h, :].astype(BF)
                    vblk = v_ref[b, rows, h, :].astype(BF)
                    s = jnp.dot(qblk, kblk.T,
                                preferred_element_type=F32) * 0.125
                    e = jnp.exp(s)
                    acc = jnp.dot(e.astype(BF), vblk,
                                  preferred_element_type=F32)
                    acc_tx[b, rows, h * Dh:(h + 1) * Dh] = acc.astype(BF)
                    l_tx[b * Hq + h, rows] = jnp.sum(e, axis=1)

        acc_rdmas, l_rdmas = [], []
        for j in range(1, N_DEV):
            tgt = lax.rem(me + j, N_DEV)
            r = pltpu.make_async_remote_copy(
                src_ref=acc_tx, dst_ref=acc_rx.at[j - 1],
                send_sem=acc_send.at[j - 1], recv_sem=acc_recv.at[j - 1],
                device_id=(tgt,), device_id_type=pl.DeviceIdType.MESH)
            r.start()
            acc_rdmas.append(r)
            r = pltpu.make_async_remote_copy(
                src_ref=l_tx, dst_ref=l_rx.at[j - 1],
                send_sem=l_send.at[j - 1], recv_sem=l_recv.at[j - 1],
                device_id=(tgt,), device_id_type=pl.DeviceIdType.MESH)
            r.start()
            l_rdmas.append(r)

        acc_tot = acc_tx[...].astype(F32)
        l_tot = l_tx[...]
        for j in range(N_DEV - 1):
            acc_rdmas[j].wait_recv()
            l_rdmas[j].wait_recv()
            acc_tot = acc_tot + acc_rx[j].astype(F32)
            l_tot = l_tot + l_rx[j]
        for j in range(N_DEV - 1):
            acc_rdmas[j].wait_send()
            l_rdmas[j].wait_send()

        for b in range(B):
            parts = []
            for h in range(Hq):
                a = acc_tot[b, :, h * Dh:(h + 1) * Dh]
                lv = l_tot[b * Hq + h]
                parts.append(a / lv[:, None])
            ctx = jnp.concatenate(parts, axis=1)
            out_ref[b] = jnp.dot(ctx.astype(BF), wo_ref[...].astype(BF),
                                 preferred_element_type=F32)

    return pl.pallas_call(
        body,
        out_shape=jax.ShapeDtypeStruct((B, Sq, d_model), F32),
        in_specs=[pl.BlockSpec(memory_space=pltpu.VMEM)] * 5,
        out_specs=pl.BlockSpec(memory_space=pltpu.VMEM),
        scratch_shapes=[
            pltpu.VMEM((B, Sq, DM), BF),
            pltpu.VMEM((B * Hq, Sq), F32),
            pltpu.VMEM((N_DEV - 1, B, Sq, DM), BF),
            pltpu.VMEM((N_DEV - 1, B * Hq, Sq), F32),
            pltpu.SemaphoreType.DMA((N_DEV - 1,)),
            pltpu.SemaphoreType.DMA((N_DEV - 1,)),
            pltpu.SemaphoreType.DMA((N_DEV - 1,)),
            pltpu.SemaphoreType.DMA((N_DEV - 1,)),
        ],
        compiler_params=pltpu.CompilerParams(collective_id=0),
    )(x, Wq, K_ext, V_ext, Wo)


# baseline (device time: 62158 ns/iter reference)
import jax
import jax.numpy as jnp
from jax import lax
from jax.experimental import pallas as pl
from jax.experimental.pallas import tpu as pltpu

N_DEV = 16
B, Sq, Hq, Dh = 2, 256, 4, 64
BLK = 64
NQB = Sq // BLK
DM = Hq * Dh
BF = jnp.bfloat16
F32 = jnp.float32


def kernel(x, Wq, K_ext, V_ext, Wo):
    d_model = x.shape[-1]

    def body(x_ref, wq_ref, k_ref, v_ref, wo_ref, out_ref,
             acc_tx, l_tx, acc_rx, l_rx,
             acc_send, acc_recv, l_send, l_recv):
        me = lax.axis_index("i")

        for b in range(B):
            q_full = jnp.dot(x_ref[b].astype(BF), wq_ref[...].astype(BF),
                             preferred_element_type=F32)
            for h in range(Hq):
                for qb in range(NQB):
                    rows = slice(qb * BLK, (qb + 1) * BLK)
                    qblk = q_full[rows, h * Dh:(h + 1) * Dh].astype(BF)
                    kblk = k_ref[b, rows, h, :].astype(BF)
                    vblk = v_ref[b, rows, h, :].astype(BF)
                    s = jnp.dot(qblk, kblk.T,
                                preferred_element_type=F32) * 0.125
                    e = jnp.exp(s)
                    acc = jnp.dot(e.astype(BF), vblk,
                                  preferred_element_type=F32)
                    acc_tx[b, rows, h * Dh:(h + 1) * Dh] = acc.astype(BF)
                    l_tx[b * Hq + h, rows] = jnp.sum(e, axis=1)

        acc_rdmas, l_rdmas = [], []
        for j in range(1, N_DEV):
            tgt = lax.rem(me + j, N_DEV)
            r = pltpu.make_async_remote_copy(
                src_ref=acc_tx, dst_ref=acc_rx.at[j - 1],
                send_sem=acc_send.at[j - 1], recv_sem=acc_recv.at[j - 1],
                device_id=(tgt,), device_id_type=pl.DeviceIdType.MESH)
            r.start()
            acc_rdmas.append(r)
            r = pltpu.make_async_remote_copy(
                src_ref=l_tx, dst_ref=l_rx.at[j - 1],
                send_sem=l_send.at[j - 1], recv_sem=l_recv.at[j - 1],
                device_id=(tgt,), device_id_type=pl.DeviceIdType.MESH)
            r.start()
            l_rdmas.append(r)

        acc_tot = acc_tx[...].astype(F32)
        l_tot = l_tx[...]
        for j in range(N_DEV - 1):
            acc_rdmas[j].wait_recv()
            l_rdmas[j].wait_recv()
            acc_tot = acc_tot + acc_rx[j].astype(F32)
            l_tot = l_tot + l_rx[j]
        for j in range(N_DEV - 1):
            acc_rdmas[j].wait_send()
            l_rdmas[j].wait_send()

        for b in range(B):
            parts = []
            for h in range(Hq):
                a = acc_tot[b, :, h * Dh:(h + 1) * Dh]
                lv = l_tot[b * Hq + h]
                parts.append(a / lv[:, None])
            ctx = jnp.concatenate(parts, axis=1)
            out_ref[b] = jnp.dot(ctx.astype(BF), wo_ref[...].astype(BF),
                                 preferred_element_type=F32)

    return pl.pallas_call(
        body,
        out_shape=jax.ShapeDtypeStruct((B, Sq, d_model), F32),
        in_specs=[pl.BlockSpec(memory_space=pltpu.VMEM)] * 5,
        out_specs=pl.BlockSpec(memory_space=pltpu.VMEM),
        scratch_shapes=[
            pltpu.VMEM((B, Sq, DM), BF),
            pltpu.VMEM((B * Hq, Sq), F32),
            pltpu.VMEM((N_DEV - 1, B, Sq, DM), BF),
            pltpu.VMEM((N_DEV - 1, B * Hq, Sq), F32),
            pltpu.SemaphoreType.DMA((N_DEV - 1,)),
            pltpu.SemaphoreType.DMA((N_DEV - 1,)),
            pltpu.SemaphoreType.DMA((N_DEV - 1,)),
            pltpu.SemaphoreType.DMA((N_DEV - 1,)),
        ],
    )(x, Wq, K_ext, V_ext, Wo)


# device time: 30165 ns/iter; 2.0606x vs baseline; 2.0606x over previous
import jax
import jax.numpy as jnp
from jax import lax
from jax.experimental import pallas as pl
from jax.experimental.pallas import tpu as pltpu

N_DEV = 16
B, Sq, Hq, Dh = 2, 256, 4, 64
BLK = 64
NQB = Sq // BLK
DM = Hq * Dh
R = Sq // N_DEV
BF = jnp.bfloat16
F32 = jnp.float32


def kernel(x, Wq, K_ext, V_ext, Wo):
    d_model = x.shape[-1]

    def body(x_ref, wq_ref, k_ref, v_ref, wo_ref, out_ref,
             acc_tx, l_tx, acc1_rx, l1_rx, ctx_buf,
             a1_send, a1_recv, l1_send, l1_recv, s2_send, s2_recv):
        me = lax.axis_index("i")

        for b in range(B):
            q_full = jnp.dot(x_ref[b].astype(BF), wq_ref[...].astype(BF),
                             preferred_element_type=F32)
            for h in range(Hq):
                for qb in range(NQB):
                    rows = slice(qb * BLK, (qb + 1) * BLK)
                    qblk = q_full[rows, h * Dh:(h + 1) * Dh].astype(BF)
                    kblk = k_ref[b, rows, h, :].astype(BF)
                    vblk = v_ref[b, rows, h, :].astype(BF)
                    s = jnp.dot(qblk, kblk.T,
                                preferred_element_type=F32) * 0.125
                    e = jnp.exp(s)
                    acc = jnp.dot(e.astype(BF), vblk,
                                  preferred_element_type=F32)
                    acc_tx[b, rows, h * Dh:(h + 1) * Dh] = acc.astype(BF)
                    l_tx[rows, b * Hq + h] = jnp.sum(e, axis=1)

        p1 = []
        for j in range(1, N_DEV):
            tgt = lax.rem(me + j, N_DEV)
            r = pltpu.make_async_remote_copy(
                src_ref=acc_tx.at[:, pl.ds(tgt * R, R), :],
                dst_ref=acc1_rx.at[j - 1],
                send_sem=a1_send.at[j - 1], recv_sem=a1_recv.at[j - 1],
                device_id=(tgt,), device_id_type=pl.DeviceIdType.MESH)
            r.start()
            p1.append(r)
            r = pltpu.make_async_remote_copy(
                src_ref=l_tx.at[pl.ds(tgt * R, R), :],
                dst_ref=l1_rx.at[j - 1],
                send_sem=l1_send.at[j - 1], recv_sem=l1_recv.at[j - 1],
                device_id=(tgt,), device_id_type=pl.DeviceIdType.MESH)
            r.start()
            p1.append(r)

        acc_slc = acc_tx[:, pl.ds(me * R, R), :].astype(F32)
        l_slc = l_tx[pl.ds(me * R, R), :]
        for r in p1:
            r.wait_recv()
        for j in range(N_DEV - 1):
            acc_slc = acc_slc + acc1_rx[j].astype(F32)
            l_slc = l_slc + l1_rx[j]

        for b in range(B):
            parts = []
            for h in range(Hq):
                a = acc_slc[b, :, h * Dh:(h + 1) * Dh]
                lv = l_slc[:, b * Hq + h]
                parts.append(a / lv[:, None])
            ctx_buf[b, pl.ds(me * R, R), :] = jnp.concatenate(
                parts, axis=1).astype(BF)

        p2 = []
        for j in range(1, N_DEV):
            tgt = lax.rem(me + j, N_DEV)
            r = pltpu.make_async_remote_copy(
                src_ref=ctx_buf.at[:, pl.ds(me * R, R), :],
                dst_ref=ctx_buf.at[:, pl.ds(me * R, R), :],
                send_sem=s2_send.at[j - 1], recv_sem=s2_recv.at[j - 1],
                device_id=(tgt,), device_id_type=pl.DeviceIdType.MESH)
            r.start()
            p2.append(r)
        for r in p2:
            r.wait_recv()
        for r in p1:
            r.wait_send()
        for r in p2:
            r.wait_send()

        for b in range(B):
            out_ref[b] = jnp.dot(ctx_buf[b].astype(BF),
                                 wo_ref[...].astype(BF),
                                 preferred_element_type=F32)

    return pl.pallas_call(
        body,
        out_shape=jax.ShapeDtypeStruct((B, Sq, d_model), F32),
        in_specs=[pl.BlockSpec(memory_space=pltpu.VMEM)] * 5,
        out_specs=pl.BlockSpec(memory_space=pltpu.VMEM),
        scratch_shapes=[
            pltpu.VMEM((B, Sq, DM), BF),
            pltpu.VMEM((Sq, B * Hq), F32),
            pltpu.VMEM((N_DEV - 1, B, R, DM), BF),
            pltpu.VMEM((N_DEV - 1, R, B * Hq), F32),
            pltpu.VMEM((B, Sq, DM), BF),
            pltpu.SemaphoreType.DMA((N_DEV - 1,)),
            pltpu.SemaphoreType.DMA((N_DEV - 1,)),
            pltpu.SemaphoreType.DMA((N_DEV - 1,)),
            pltpu.SemaphoreType.DMA((N_DEV - 1,)),
            pltpu.SemaphoreType.DMA((N_DEV - 1,)),
            pltpu.SemaphoreType.DMA((N_DEV - 1,)),
        ],
    )(x, Wq, K_ext, V_ext, Wo)


# device time: 28795 ns/iter; 2.1586x vs baseline; 1.0476x over previous
import jax
import jax.numpy as jnp
from jax import lax
from jax.experimental import pallas as pl
from jax.experimental.pallas import tpu as pltpu

N_DEV = 16
B, Sq, Hq, Dh = 2, 256, 4, 64
BLK = 64
NQB = Sq // BLK
DM = Hq * Dh
R = Sq // N_DEV
CW = B * DM + B * Hq
BF = jnp.bfloat16
F32 = jnp.float32


def kernel(x, Wq, K_ext, V_ext, Wo):
    d_model = x.shape[-1]

    def body(x_ref, wq_ref, k_ref, v_ref, wo_ref, out_ref,
             comb_tx, comb_rx, ctx_buf,
             s1_send, s1_recv, s2_send, s2_recv):
        me = lax.axis_index("i")

        for b in range(B):
            q_full = jnp.dot(x_ref[b].astype(BF), wq_ref[...].astype(BF),
                             preferred_element_type=F32)
            for h in range(Hq):
                for qb in range(NQB):
                    rows = slice(qb * BLK, (qb + 1) * BLK)
                    qblk = q_full[rows, h * Dh:(h + 1) * Dh].astype(BF)
                    kblk = k_ref[b, rows, h, :].astype(BF)
                    vblk = v_ref[b, rows, h, :].astype(BF)
                    s = jnp.dot(qblk, kblk.T,
                                preferred_element_type=F32) * 0.125
                    e = jnp.exp(s)
                    acc = jnp.dot(e.astype(BF), vblk,
                                  preferred_element_type=F32)
                    col = b * DM + h * Dh
                    comb_tx[rows, col:col + Dh] = acc.astype(BF)
                    lcol = B * DM + b * Hq + h
                    comb_tx[rows, lcol:lcol + 1] = jnp.sum(
                        e, axis=1, keepdims=True).astype(BF)

        p1 = []
        for j in range(1, N_DEV):
            tgt = lax.rem(me + j, N_DEV)
            r = pltpu.make_async_remote_copy(
                src_ref=comb_tx.at[pl.ds(tgt * R, R), :],
                dst_ref=comb_rx.at[j - 1],
                send_sem=s1_send.at[j - 1], recv_sem=s1_recv.at[j - 1],
                device_id=(tgt,), device_id_type=pl.DeviceIdType.MESH)
            r.start()
            p1.append(r)

        slc = comb_tx[pl.ds(me * R, R), :].astype(F32)
        for r in p1:
            r.wait_recv()
        for j in range(N_DEV - 1):
            slc = slc + comb_rx[j].astype(F32)

        for b in range(B):
            parts = []
            for h in range(Hq):
                a = slc[:, b * DM + h * Dh:b * DM + (h + 1) * Dh]
                lv = slc[:, B * DM + b * Hq + h]
                parts.append(a / lv[:, None])
            ctx_buf[b, pl.ds(me * R, R), :] = jnp.concatenate(
                parts, axis=1).astype(BF)

        p2 = []
        for j in range(1, N_DEV):
            tgt = lax.rem(me + j, N_DEV)
            r = pltpu.make_async_remote_copy(
                src_ref=ctx_buf.at[:, pl.ds(me * R, R), :],
                dst_ref=ctx_buf.at[:, pl.ds(me * R, R), :],
                send_sem=s2_send.at[j - 1], recv_sem=s2_recv.at[j - 1],
                device_id=(tgt,), device_id_type=pl.DeviceIdType.MESH)
            r.start()
            p2.append(r)
        for r in p2:
            r.wait_recv()
        for r in p1:
            r.wait_send()
        for r in p2:
            r.wait_send()

        for b in range(B):
            out_ref[b] = jnp.dot(ctx_buf[b].astype(BF),
                                 wo_ref[...].astype(BF),
                                 preferred_element_type=F32)

    return pl.pallas_call(
        body,
        out_shape=jax.ShapeDtypeStruct((B, Sq, d_model), F32),
        in_specs=[pl.BlockSpec(memory_space=pltpu.VMEM)] * 5,
        out_specs=pl.BlockSpec(memory_space=pltpu.VMEM),
        scratch_shapes=[
            pltpu.VMEM((Sq, CW), BF),
            pltpu.VMEM((N_DEV - 1, R, CW), BF),
            pltpu.VMEM((B, Sq, DM), BF),
            pltpu.SemaphoreType.DMA((N_DEV - 1,)),
            pltpu.SemaphoreType.DMA((N_DEV - 1,)),
            pltpu.SemaphoreType.DMA((N_DEV - 1,)),
            pltpu.SemaphoreType.DMA((N_DEV - 1,)),
        ],
    )(x, Wq, K_ext, V_ext, Wo)


# device time: 28594 ns/iter; 2.1738x vs baseline; 1.0070x over previous
import jax
import jax.numpy as jnp
from jax import lax
from jax.experimental import pallas as pl
from jax.experimental.pallas import tpu as pltpu

N_DEV = 16
B, Sq, Hq, Dh = 2, 256, 4, 64
BLK = 64
NQB = Sq // BLK
DM = Hq * Dh
R = Sq // N_DEV
CW = B * DM + B * Hq
BF = jnp.bfloat16
F32 = jnp.float32


def kernel(x, Wq, K_ext, V_ext, Wo):
    d_model = x.shape[-1]
    K2 = K_ext.reshape(B, Sq, DM)
    V2 = V_ext.reshape(B, Sq, DM)

    def body(x_ref, wq_ref, k_ref, v_ref, wo_ref, out_ref,
             comb_tx, comb_rx, ctx_buf,
             s1_send, s1_recv, s2_send, s2_recv):
        me = lax.axis_index("i")

        import os
        _scope = (jax.named_scope if os.environ.get("KPROF")
                  else (lambda name: __import__("contextlib").nullcontext()))
        with _scope("compute_partial"):
         for b in range(B):
            q_full = jnp.dot(x_ref[b].astype(BF), wq_ref[...].astype(BF),
                             preferred_element_type=F32)
            for h in range(Hq):
                for qb in range(NQB):
                    rows = slice(qb * BLK, (qb + 1) * BLK)
                    qblk = q_full[rows, h * Dh:(h + 1) * Dh].astype(BF)
                    kblk = k_ref[b, rows, h * Dh:(h + 1) * Dh].astype(BF)
                    vblk = v_ref[b, rows, h * Dh:(h + 1) * Dh].astype(BF)
                    s = jnp.dot(qblk, kblk.T,
                                preferred_element_type=F32) * 0.125
                    e = jnp.exp(s)
                    acc = jnp.dot(e.astype(BF), vblk,
                                  preferred_element_type=F32)
                    col = b * DM + h * Dh
                    comb_tx[rows, col:col + Dh] = acc.astype(BF)
                    lcol = B * DM + b * Hq + h
                    comb_tx[rows, lcol:lcol + 1] = jnp.sum(
                        e, axis=1, keepdims=True).astype(BF)

        p1 = []
        with _scope("p1_issue"):
         for j in range(1, N_DEV):
            tgt = lax.rem(me + j, N_DEV)
            r = pltpu.make_async_remote_copy(
                src_ref=comb_tx.at[pl.ds(tgt * R, R), :],
                dst_ref=comb_rx.at[j - 1],
                send_sem=s1_send.at[j - 1], recv_sem=s1_recv.at[j - 1],
                device_id=(tgt,), device_id_type=pl.DeviceIdType.MESH)
            r.start()
            p1.append(r)

        slc = comb_tx[pl.ds(me * R, R), :].astype(F32)
        with _scope("p1_wait"):
         for r in p1:
            r.wait_recv()
        with _scope("reduce_norm"):
         for j in range(N_DEV - 1):
            slc = slc + comb_rx[j].astype(F32)

         for b in range(B):
            parts = []
            for h in range(Hq):
                a = slc[:, b * DM + h * Dh:b * DM + (h + 1) * Dh]
                lv = slc[:, B * DM + b * Hq + h]
                parts.append(a / lv[:, None])
            ctx_buf[b, pl.ds(me * R, R), :] = jnp.concatenate(
                parts, axis=1).astype(BF)

        p2 = []
        with _scope("p2_issue"):
         for j in range(1, N_DEV):
            tgt = lax.rem(me + j, N_DEV)
            r = pltpu.make_async_remote_copy(
                src_ref=ctx_buf.at[:, pl.ds(me * R, R), :],
                dst_ref=ctx_buf.at[:, pl.ds(me * R, R), :],
                send_sem=s2_send.at[j - 1], recv_sem=s2_recv.at[j - 1],
                device_id=(tgt,), device_id_type=pl.DeviceIdType.MESH)
            r.start()
            p2.append(r)
        with _scope("p2_wait"):
         for r in p2:
            r.wait_recv()
         for r in p1:
            r.wait_send()
         for r in p2:
            r.wait_send()

        with _scope("wo_proj"):
         for b in range(B):
            out_ref[b] = jnp.dot(ctx_buf[b].astype(BF),
                                 wo_ref[...].astype(BF),
                                 preferred_element_type=F32)

    return pl.pallas_call(
        body,
        out_shape=jax.ShapeDtypeStruct((B, Sq, d_model), F32),
        in_specs=[pl.BlockSpec(memory_space=pltpu.VMEM)] * 5,
        out_specs=pl.BlockSpec(memory_space=pltpu.VMEM),
        scratch_shapes=[
            pltpu.VMEM((Sq, CW), BF),
            pltpu.VMEM((N_DEV - 1, R, CW), BF),
            pltpu.VMEM((B, Sq, DM), BF),
            pltpu.SemaphoreType.DMA((N_DEV - 1,)),
            pltpu.SemaphoreType.DMA((N_DEV - 1,)),
            pltpu.SemaphoreType.DMA((N_DEV - 1,)),
            pltpu.SemaphoreType.DMA((N_DEV - 1,)),
        ],
    )(x, Wq, K2, V2, Wo)


# device time: 22322 ns/iter; 2.7846x vs baseline; 1.2810x over previous
import jax
import jax.numpy as jnp
from jax import lax
from jax.experimental import pallas as pl
from jax.experimental.pallas import tpu as pltpu

N_DEV = 16
B, Sq, Hq, Dh = 2, 256, 4, 64
BLK = 64
NQB = Sq // BLK
DM = Hq * Dh
R = Sq // N_DEV
CW = B * DM + B * Hq
BF = jnp.bfloat16
F32 = jnp.float32


def kernel(x, Wq, K_ext, V_ext, Wo):
    d_model = x.shape[-1]
    K2 = K_ext.reshape(B, Sq, DM)
    V2 = V_ext.reshape(B, Sq, DM)

    def body(x_ref, wq_ref, k_ref, v_ref, wo_ref, out_ref,
             comb_tx, comb_rx, ctx_buf,
             s1_send, s1_recv, s2_send, s2_recv):
        me = lax.axis_index("i")

        bsem = pltpu.get_barrier_semaphore()
        pl.semaphore_signal(bsem, inc=1, device_id=(me,),
                            device_id_type=pl.DeviceIdType.MESH)
        pl.semaphore_wait(bsem, 1)

        import os
        _scope = (jax.named_scope if os.environ.get("KPROF")
                  else (lambda name: __import__("contextlib").nullcontext()))
        with _scope("compute_partial"):
         for b in range(B):
            q_full = jnp.dot(x_ref[b].astype(BF), wq_ref[...].astype(BF),
                             preferred_element_type=F32)
            for h in range(Hq):
                for qb in range(NQB):
                    rows = slice(qb * BLK, (qb + 1) * BLK)
                    qblk = q_full[rows, h * Dh:(h + 1) * Dh].astype(BF)
                    kblk = k_ref[b, rows, h * Dh:(h + 1) * Dh].astype(BF)
                    vblk = v_ref[b, rows, h * Dh:(h + 1) * Dh].astype(BF)
                    s = jnp.dot(qblk, kblk.T,
                                preferred_element_type=F32) * 0.125
                    e = jnp.exp(s)
                    acc = jnp.dot(e.astype(BF), vblk,
                                  preferred_element_type=F32)
                    col = b * DM + h * Dh
                    comb_tx[rows, col:col + Dh] = acc.astype(BF)
                    lcol = B * DM + b * Hq + h
                    comb_tx[rows, lcol:lcol + 1] = jnp.sum(
                        e, axis=1, keepdims=True).astype(BF)

        p1 = []
        with _scope("p1_issue"):
         for j in range(1, N_DEV):
            tgt = lax.rem(me + j, N_DEV)
            r = pltpu.make_async_remote_copy(
                src_ref=comb_tx.at[pl.ds(tgt * R, R), :],
                dst_ref=comb_rx.at[j - 1],
                send_sem=s1_send.at[j - 1], recv_sem=s1_recv.at[j - 1],
                device_id=(tgt,), device_id_type=pl.DeviceIdType.MESH)
            r.start()
            p1.append(r)

        slc = comb_tx[pl.ds(me * R, R), :].astype(F32)
        with _scope("p1_wait"):
         for r in p1:
            r.wait_recv()
        with _scope("reduce_norm"):
         for j in range(N_DEV - 1):
            slc = slc + comb_rx[j].astype(F32)

         for b in range(B):
            parts = []
            for h in range(Hq):
                a = slc[:, b * DM + h * Dh:b * DM + (h + 1) * Dh]
                lv = slc[:, B * DM + b * Hq + h]
                parts.append(a / lv[:, None])
            ctx_buf[b, pl.ds(me * R, R), :] = jnp.concatenate(
                parts, axis=1).astype(BF)

        p2 = []
        with _scope("p2_issue"):
         for j in range(1, N_DEV):
            tgt = lax.rem(me + j, N_DEV)
            r = pltpu.make_async_remote_copy(
                src_ref=ctx_buf.at[:, pl.ds(me * R, R), :],
                dst_ref=ctx_buf.at[:, pl.ds(me * R, R), :],
                send_sem=s2_send.at[j - 1], recv_sem=s2_recv.at[j - 1],
                device_id=(tgt,), device_id_type=pl.DeviceIdType.MESH)
            r.start()
            p2.append(r)
        with _scope("p2_wait"):
         for r in p2:
            r.wait_recv()
         for r in p1:
            r.wait_send()
         for r in p2:
            r.wait_send()

        with _scope("wo_proj"):
         for b in range(B):
            out_ref[b] = jnp.dot(ctx_buf[b].astype(BF),
                                 wo_ref[...].astype(BF),
                                 preferred_element_type=F32)

    return pl.pallas_call(
        body,
        out_shape=jax.ShapeDtypeStruct((B, Sq, d_model), F32),
        in_specs=[pl.BlockSpec(memory_space=pltpu.VMEM)] * 5,
        out_specs=pl.BlockSpec(memory_space=pltpu.VMEM),
        scratch_shapes=[
            pltpu.VMEM((Sq, CW), BF),
            pltpu.VMEM((N_DEV - 1, R, CW), BF),
            pltpu.VMEM((B, Sq, DM), BF),
            pltpu.SemaphoreType.DMA((N_DEV - 1,)),
            pltpu.SemaphoreType.DMA((N_DEV - 1,)),
            pltpu.SemaphoreType.DMA((N_DEV - 1,)),
            pltpu.SemaphoreType.DMA((N_DEV - 1,)),
        ],
        compiler_params=pltpu.CompilerParams(collective_id=0),
    )(x, Wq, K2, V2, Wo)


# device time: 19490 ns/iter; 3.1892x vs baseline; 1.1453x over previous
import jax
import jax.numpy as jnp
from jax import lax
from jax.experimental import pallas as pl
from jax.experimental.pallas import tpu as pltpu

N_DEV = 16
B, Sq, Hq, Dh = 2, 256, 4, 64
BLK = 64
NQB = Sq // BLK
DM = Hq * Dh
R = Sq // N_DEV
CW = B * DM + B * Hq
BF = jnp.bfloat16
F32 = jnp.float32


def kernel(x, Wq, K_ext, V_ext, Wo):
    d_model = x.shape[-1]
    K2 = K_ext.reshape(B, Sq, DM)
    V2 = V_ext.reshape(B, Sq, DM)

    def body(x_ref, wq_ref, k_ref, v_ref, wo_ref, out_ref,
             comb_tx, comb_rx, ctx_buf,
             s1_send, s1_recv, s2_send, s2_recv):
        me = lax.axis_index("i")

        bsem = pltpu.get_barrier_semaphore()
        pl.semaphore_signal(bsem, inc=1, device_id=(me,),
                            device_id_type=pl.DeviceIdType.MESH)
        pl.semaphore_wait(bsem, 1)

        import os
        _scope = (jax.named_scope if os.environ.get("KPROF")
                  else (lambda name: __import__("contextlib").nullcontext()))
        ri = lax.broadcasted_iota(jnp.int32, (Sq, Sq), 0) // BLK
        ci = lax.broadcasted_iota(jnp.int32, (Sq, Sq), 1) // BLK
        bd_mask = ri == ci
        with _scope("compute_partial"):
         for b in range(B):
            q_full = jnp.dot(x_ref[b].astype(BF), wq_ref[...].astype(BF),
                             preferred_element_type=F32)
            for h in range(Hq):
                cols = slice(h * Dh, (h + 1) * Dh)
                qh = q_full[:, cols].astype(BF)
                kh = k_ref[b, :, cols].astype(BF)
                vh = v_ref[b, :, cols].astype(BF)
                s = jnp.dot(qh, kh.T,
                            preferred_element_type=F32) * 0.125
                e = jnp.where(bd_mask, jnp.exp(s), 0.0)
                acc = jnp.dot(e.astype(BF), vh,
                              preferred_element_type=F32)
                comb_tx[:, b * DM + h * Dh:b * DM + (h + 1) * Dh] = (
                    acc.astype(BF))
                lcol = B * DM + b * Hq + h
                comb_tx[:, lcol:lcol + 1] = jnp.sum(
                    e, axis=1, keepdims=True).astype(BF)

        p1 = []
        with _scope("p1_issue"):
         for j in range(1, N_DEV):
            tgt = lax.rem(me + j, N_DEV)
            r = pltpu.make_async_remote_copy(
                src_ref=comb_tx.at[pl.ds(tgt * R, R), :],
                dst_ref=comb_rx.at[j - 1],
                send_sem=s1_send.at[j - 1], recv_sem=s1_recv.at[j - 1],
                device_id=(tgt,), device_id_type=pl.DeviceIdType.MESH)
            r.start()
            p1.append(r)

        slc = comb_tx[pl.ds(me * R, R), :].astype(F32)
        with _scope("p1_wait"):
         for r in p1:
            r.wait_recv()
        with _scope("reduce_norm"):
         for j in range(N_DEV - 1):
            slc = slc + comb_rx[j].astype(F32)

         for b in range(B):
            parts = []
            for h in range(Hq):
                a = slc[:, b * DM + h * Dh:b * DM + (h + 1) * Dh]
                lv = slc[:, B * DM + b * Hq + h]
                parts.append(a / lv[:, None])
            ctx_buf[b, pl.ds(me * R, R), :] = jnp.concatenate(
                parts, axis=1).astype(BF)

        p2 = []
        with _scope("p2_issue"):
         for j in range(1, N_DEV):
            tgt = lax.rem(me + j, N_DEV)
            r = pltpu.make_async_remote_copy(
                src_ref=ctx_buf.at[:, pl.ds(me * R, R), :],
                dst_ref=ctx_buf.at[:, pl.ds(me * R, R), :],
                send_sem=s2_send.at[j - 1], recv_sem=s2_recv.at[j - 1],
                device_id=(tgt,), device_id_type=pl.DeviceIdType.MESH)
            r.start()
            p2.append(r)
        with _scope("p2_wait"):
         for r in p2:
            r.wait_recv()
         for r in p1:
            r.wait_send()
         for r in p2:
            r.wait_send()

        with _scope("wo_proj"):
         for b in range(B):
            out_ref[b] = jnp.dot(ctx_buf[b].astype(BF),
                                 wo_ref[...].astype(BF),
                                 preferred_element_type=F32)

    return pl.pallas_call(
        body,
        out_shape=jax.ShapeDtypeStruct((B, Sq, d_model), F32),
        in_specs=[pl.BlockSpec(memory_space=pltpu.VMEM)] * 5,
        out_specs=pl.BlockSpec(memory_space=pltpu.VMEM),
        scratch_shapes=[
            pltpu.VMEM((Sq, CW), BF),
            pltpu.VMEM((N_DEV - 1, R, CW), BF),
            pltpu.VMEM((B, Sq, DM), BF),
            pltpu.SemaphoreType.DMA((N_DEV - 1,)),
            pltpu.SemaphoreType.DMA((N_DEV - 1,)),
            pltpu.SemaphoreType.DMA((N_DEV - 1,)),
            pltpu.SemaphoreType.DMA((N_DEV - 1,)),
        ],
        compiler_params=pltpu.CompilerParams(collective_id=0),
    )(x, Wq, K2, V2, Wo)
